# baseline (speedup 1.0000x reference)
import functools

import jax
import jax.numpy as jnp
from jax.experimental import pallas as pl
from jax.experimental.pallas import tpu as pltpu
from jax.experimental.pallas import tpu_sc as plsc

_IGNORE = 255
_THRESH = 0.7
_K = 131072

_B, _C, _H, _W = 8, 19, 512, 512
_RB = 128
_NR = _H // _RB

_TOP_BITS = 0x3F800000


def _pixel_stats(pred_ref, tgt_ref):
    x = pred_ref[0]
    t = tgt_ref[0]
    m = jnp.max(x, axis=0)
    xs = x - m[None, :, :]
    e = jnp.exp(xs)
    s = jnp.sum(e, axis=0)
    cls = jax.lax.broadcasted_iota(jnp.int32, (_C, _RB, _W), 0)
    onehot = cls == t[None, :, :]
    shift = jnp.sum(jnp.where(onehot, xs, 0.0), axis=0)
    valid = t != _IGNORE
    p = jnp.exp(shift) / s
    p_eff = jnp.where(valid, p, 1.0)
    nl = jnp.log(s) - shift
    snl = jnp.where(valid, nl, -1.0)
    return p_eff, snl, nl


def _main_body(pred_ref, tgt_ref, cnt_ref, sum_ref):
    p_eff, _, nl = _pixel_stats(pred_ref, tgt_ref)
    kept = p_eff <= _THRESH
    cpart = jnp.sum(kept.astype(jnp.float32), axis=0)
    spart = jnp.sum(jnp.where(kept, nl, 0.0), axis=0)
    first = (pl.program_id(0) == 0) & (pl.program_id(1) == 0)

    @pl.when(first)
    def _init():
        cnt_ref[...] = cpart[None, :]
        sum_ref[...] = spart[None, :]

    @pl.when(jnp.logical_not(first))
    def _acc():
        cnt_ref[...] = cnt_ref[...] + cpart[None, :]
        sum_ref[...] = sum_ref[...] + spart[None, :]


def _mat_body(pred_ref, tgt_ref, p_ref, nl_ref):
    p_eff, snl, _ = _pixel_stats(pred_ref, tgt_ref)
    p_ref[0] = p_eff
    nl_ref[0] = snl


_N = _B * _H * _W
_NSUB = 16
_SHARD = _N // _NSUB
_HALF = _SHARD // 2
_NCHUNK = _HALF // 16
_ROUNDS = 31


def _sc_select_body(bits_hbm, out_hbm, shard_v, cnts_v, stage_v, shared_cnts):
    cid = jax.lax.axis_index("c")
    sid = jax.lax.axis_index("s")
    base = sid * _SHARD

    def round_body(_, lohi):
        lo, hi = lohi
        mid = (lo + hi) // 2
        acc = jnp.zeros((16,), jnp.int32)
        for half in range(2):
            pltpu.sync_copy(
                bits_hbm.at[pl.ds(base + half * _HALF, _HALF)], shard_v)

            def chunk_body(j, a):
                v = shard_v[pl.ds(j * 16, 16)]
                return a + jnp.where(v <= mid, 1, 0)

            acc = jax.lax.fori_loop(0, _NCHUNK, chunk_body, acc, unroll=4)
        stage_v[...] = acc
        pltpu.sync_copy(stage_v, shared_cnts.at[sid])
        plsc.subcore_barrier()
        pltpu.sync_copy(shared_cnts, cnts_v)
        plsc.subcore_barrier()
        tot_vec = cnts_v[0]
        for i in range(1, _NSUB):
            tot_vec = tot_vec + cnts_v[i]
        total = tot_vec[0]
        for i in range(1, 16):
            total = total + tot_vec[i]
        take_low = total >= _K
        return (jnp.where(take_low, lo, mid + 1),
                jnp.where(take_low, mid, hi))

    lo, _ = jax.lax.fori_loop(
        0, _ROUNDS, round_body, (jnp.int32(0), jnp.int32(_TOP_BITS)))

    @pl.when((cid == 0) & (sid == 0))
    def _write():
        stage_v[...] = jnp.zeros((16,), jnp.int32) + lo
        pltpu.sync_copy(stage_v, out_hbm)


_sc_select = functools.partial(
    pl.kernel,
    mesh=plsc.VectorSubcoreMesh(core_axis_name="c", subcore_axis_name="s"),
    out_type=jax.ShapeDtypeStruct((16,), jnp.int32),
    scratch_types=[
        pltpu.VMEM((_HALF,), jnp.int32),
        pltpu.VMEM((_NSUB, 16), jnp.int32),
        pltpu.VMEM((16,), jnp.int32),
        pltpu.VMEM_SHARED((_NSUB, 16), jnp.int32),
    ],
)(_sc_select_body)


def _final_body(thr_ref, p_ref, nl_ref, cnt_ref, sum_ref):
    thr = thr_ref[0]
    p = p_ref[0]
    v = nl_ref[0]
    kept = (p <= thr) & (v >= -0.5)
    cpart = jnp.sum(kept.astype(jnp.float32), axis=0)
    spart = jnp.sum(jnp.where(kept, v, 0.0), axis=0)
    first = pl.program_id(0) == 0

    @pl.when(first)
    def _init():
        cnt_ref[...] = cpart[None, :]
        sum_ref[...] = spart[None, :]

    @pl.when(jnp.logical_not(first))
    def _acc():
        cnt_ref[...] = cnt_ref[...] + cpart[None, :]
        sum_ref[...] = sum_ref[...] + spart[None, :]


_main_call = pl.pallas_call(
    _main_body,
    grid=(_B, _NR),
    in_specs=[
        pl.BlockSpec((1, _C, _RB, _W), lambda i, j: (i, 0, j, 0)),
        pl.BlockSpec((1, _RB, _W), lambda i, j: (i, j, 0)),
    ],
    out_specs=[
        pl.BlockSpec((1, _W), lambda i, j: (0, 0)),
        pl.BlockSpec((1, _W), lambda i, j: (0, 0)),
    ],
    out_shape=[
        jax.ShapeDtypeStruct((1, _W), jnp.float32),
        jax.ShapeDtypeStruct((1, _W), jnp.float32),
    ],
)

_mat_call = pl.pallas_call(
    _mat_body,
    grid=(_B, _NR),
    in_specs=[
        pl.BlockSpec((1, _C, _RB, _W), lambda i, j: (i, 0, j, 0)),
        pl.BlockSpec((1, _RB, _W), lambda i, j: (i, j, 0)),
    ],
    out_specs=[
        pl.BlockSpec((1, _RB, _W), lambda i, j: (i, j, 0)),
        pl.BlockSpec((1, _RB, _W), lambda i, j: (i, j, 0)),
    ],
    out_shape=[
        jax.ShapeDtypeStruct((_B, _H, _W), jnp.float32),
        jax.ShapeDtypeStruct((_B, _H, _W), jnp.float32),
    ],
)

_final_call = pl.pallas_call(
    _final_body,
    grid=(_B,),
    in_specs=[
        pl.BlockSpec(memory_space=pltpu.SMEM),
        pl.BlockSpec((1, _H, _W), lambda i: (i, 0, 0)),
        pl.BlockSpec((1, _H, _W), lambda i: (i, 0, 0)),
    ],
    out_specs=[
        pl.BlockSpec((1, _W), lambda i: (0, 0)),
        pl.BlockSpec((1, _W), lambda i: (0, 0)),
    ],
    out_shape=[
        jax.ShapeDtypeStruct((1, _W), jnp.float32),
        jax.ShapeDtypeStruct((1, _W), jnp.float32),
    ],
)


def kernel(pred, target):
    cl, sl = _main_call(pred, target)
    cnt07 = jnp.sum(cl)
    sum07 = jnp.sum(sl)

    def common():
        return sum07 / jnp.maximum(cnt07, 1.0)

    def rare():
        p_eff, snl = _mat_call(pred, target)
        bits = jax.lax.bitcast_convert_type(p_eff, jnp.int32).reshape(_N)
        kth_bits = _sc_select(bits)[0]
        kth = jax.lax.bitcast_convert_type(kth_bits, jnp.float32)
        thr = jnp.maximum(kth, jnp.float32(_THRESH))

        cf, sf = _final_call(thr.reshape(1), p_eff, snl)
        return jnp.sum(sf) / jnp.maximum(jnp.sum(cf), 1.0)

    return jax.lax.cond(cnt07 >= _K, common, rare)


# speedup vs baseline: 1.0169x; 1.0169x over previous
import functools

import jax
import jax.numpy as jnp
from jax.experimental import pallas as pl
from jax.experimental.pallas import tpu as pltpu
from jax.experimental.pallas import tpu_sc as plsc

_IGNORE = 255
_THRESH = 0.7
_K = 131072

_B, _C, _H, _W = 8, 19, 512, 512
_RB = 128
_NR = _H // _RB

_TOP_BITS = 0x3F800000


def _pixel_stats(pred_ref, tgt_ref):
    x = pred_ref[0]
    t = tgt_ref[0]
    m = jnp.max(x, axis=0)
    xs = x - m[None, :, :]
    e = jnp.exp(xs)
    s = jnp.sum(e, axis=0)
    cls = jax.lax.broadcasted_iota(jnp.int32, (_C, _RB, _W), 0)
    onehot = cls == t[None, :, :]
    shift = jnp.sum(jnp.where(onehot, xs, 0.0), axis=0)
    valid = t != _IGNORE
    p = jnp.exp(shift) / s
    p_eff = jnp.where(valid, p, 1.0)
    nl = jnp.log(s) - shift
    snl = jnp.where(valid, nl, -1.0)
    return p_eff, snl, nl


def _main_body(pred_ref, tgt_ref, cnt_ref, sum_ref, cacc, sacc):
    p_eff, _, nl = _pixel_stats(pred_ref, tgt_ref)
    kept = p_eff <= _THRESH
    cpart = jnp.sum(kept.astype(jnp.float32), axis=0)
    spart = jnp.sum(jnp.where(kept, nl, 0.0), axis=0)
    first = (pl.program_id(0) == 0) & (pl.program_id(1) == 0)
    last = ((pl.program_id(0) == _B - 1)
            & (pl.program_id(1) == _NR - 1))

    @pl.when(first)
    def _init():
        cacc[...] = cpart[None, :]
        sacc[...] = spart[None, :]

    @pl.when(jnp.logical_not(first))
    def _acc():
        cacc[...] = cacc[...] + cpart[None, :]
        sacc[...] = sacc[...] + spart[None, :]

    @pl.when(last)
    def _emit():
        cnt_ref[0, 0] = jnp.sum(cacc[...])
        sum_ref[0, 0] = jnp.sum(sacc[...])


def _mat_body(pred_ref, tgt_ref, p_ref, nl_ref):
    p_eff, snl, _ = _pixel_stats(pred_ref, tgt_ref)
    p_ref[0] = p_eff
    nl_ref[0] = snl


_N = _B * _H * _W
_NSUB = 16
_SHARD = _N // _NSUB
_HALF = _SHARD // 2
_NCHUNK = _HALF // 16
_ROUNDS = 31


def _sc_select_body(bits_hbm, out_hbm, shard_v, cnts_v, stage_v, shared_cnts):
    cid = jax.lax.axis_index("c")
    sid = jax.lax.axis_index("s")
    base = sid * _SHARD

    def round_body(_, lohi):
        lo, hi = lohi
        mid = (lo + hi) // 2
        acc = jnp.zeros((16,), jnp.int32)
        for half in range(2):
            pltpu.sync_copy(
                bits_hbm.at[pl.ds(base + half * _HALF, _HALF)], shard_v)

            def chunk_body(j, a):
                v = shard_v[pl.ds(j * 16, 16)]
                return a + jnp.where(v <= mid, 1, 0)

            acc = jax.lax.fori_loop(0, _NCHUNK, chunk_body, acc, unroll=4)
        stage_v[...] = acc
        pltpu.sync_copy(stage_v, shared_cnts.at[sid])
        plsc.subcore_barrier()
        pltpu.sync_copy(shared_cnts, cnts_v)
        plsc.subcore_barrier()
        tot_vec = cnts_v[0]
        for i in range(1, _NSUB):
            tot_vec = tot_vec + cnts_v[i]
        total = tot_vec[0]
        for i in range(1, 16):
            total = total + tot_vec[i]
        take_low = total >= _K
        return (jnp.where(take_low, lo, mid + 1),
                jnp.where(take_low, mid, hi))

    lo, _ = jax.lax.fori_loop(
        0, _ROUNDS, round_body, (jnp.int32(0), jnp.int32(_TOP_BITS)))

    @pl.when((cid == 0) & (sid == 0))
    def _write():
        stage_v[...] = jnp.zeros((16,), jnp.int32) + lo
        pltpu.sync_copy(stage_v, out_hbm)


_sc_select = functools.partial(
    pl.kernel,
    mesh=plsc.VectorSubcoreMesh(core_axis_name="c", subcore_axis_name="s"),
    out_type=jax.ShapeDtypeStruct((16,), jnp.int32),
    scratch_types=[
        pltpu.VMEM((_HALF,), jnp.int32),
        pltpu.VMEM((_NSUB, 16), jnp.int32),
        pltpu.VMEM((16,), jnp.int32),
        pltpu.VMEM_SHARED((_NSUB, 16), jnp.int32),
    ],
)(_sc_select_body)


def _final_body(thr_ref, p_ref, nl_ref, cnt_ref, sum_ref):
    thr = thr_ref[0]
    p = p_ref[0]
    v = nl_ref[0]
    kept = (p <= thr) & (v >= -0.5)
    cpart = jnp.sum(kept.astype(jnp.float32), axis=0)
    spart = jnp.sum(jnp.where(kept, v, 0.0), axis=0)
    first = pl.program_id(0) == 0

    @pl.when(first)
    def _init():
        cnt_ref[...] = cpart[None, :]
        sum_ref[...] = spart[None, :]

    @pl.when(jnp.logical_not(first))
    def _acc():
        cnt_ref[...] = cnt_ref[...] + cpart[None, :]
        sum_ref[...] = sum_ref[...] + spart[None, :]


_main_call = pl.pallas_call(
    _main_body,
    grid=(_B, _NR),
    in_specs=[
        pl.BlockSpec((1, _C, _RB, _W), lambda i, j: (i, 0, j, 0)),
        pl.BlockSpec((1, _RB, _W), lambda i, j: (i, j, 0)),
    ],
    out_specs=[
        pl.BlockSpec(memory_space=pltpu.SMEM),
        pl.BlockSpec(memory_space=pltpu.SMEM),
    ],
    out_shape=[
        jax.ShapeDtypeStruct((1, 1), jnp.float32),
        jax.ShapeDtypeStruct((1, 1), jnp.float32),
    ],
    scratch_shapes=[
        pltpu.VMEM((1, _W), jnp.float32),
        pltpu.VMEM((1, _W), jnp.float32),
    ],
)

_mat_call = pl.pallas_call(
    _mat_body,
    grid=(_B, _NR),
    in_specs=[
        pl.BlockSpec((1, _C, _RB, _W), lambda i, j: (i, 0, j, 0)),
        pl.BlockSpec((1, _RB, _W), lambda i, j: (i, j, 0)),
    ],
    out_specs=[
        pl.BlockSpec((1, _RB, _W), lambda i, j: (i, j, 0)),
        pl.BlockSpec((1, _RB, _W), lambda i, j: (i, j, 0)),
    ],
    out_shape=[
        jax.ShapeDtypeStruct((_B, _H, _W), jnp.float32),
        jax.ShapeDtypeStruct((_B, _H, _W), jnp.float32),
    ],
)

_final_call = pl.pallas_call(
    _final_body,
    grid=(_B,),
    in_specs=[
        pl.BlockSpec(memory_space=pltpu.SMEM),
        pl.BlockSpec((1, _H, _W), lambda i: (i, 0, 0)),
        pl.BlockSpec((1, _H, _W), lambda i: (i, 0, 0)),
    ],
    out_specs=[
        pl.BlockSpec((1, _W), lambda i: (0, 0)),
        pl.BlockSpec((1, _W), lambda i: (0, 0)),
    ],
    out_shape=[
        jax.ShapeDtypeStruct((1, _W), jnp.float32),
        jax.ShapeDtypeStruct((1, _W), jnp.float32),
    ],
)


def kernel(pred, target):
    cl, sl = _main_call(pred, target)
    cnt07 = cl[0, 0]
    sum07 = sl[0, 0]

    def common():
        return sum07 / jnp.maximum(cnt07, 1.0)

    def rare():
        p_eff, snl = _mat_call(pred, target)
        bits = jax.lax.bitcast_convert_type(p_eff, jnp.int32).reshape(_N)
        kth_bits = _sc_select(bits)[0]
        kth = jax.lax.bitcast_convert_type(kth_bits, jnp.float32)
        thr = jnp.maximum(kth, jnp.float32(_THRESH))

        cf, sf = _final_call(thr.reshape(1), p_eff, snl)
        return jnp.sum(sf) / jnp.maximum(jnp.sum(cf), 1.0)

    return jax.lax.cond(cnt07 >= _K, common, rare)
